# Initial kernel scaffold; baseline (speedup 1.0000x reference)
import jax
import jax.numpy as jnp
from jax.experimental import pallas as pl


def kernel(x, inner_position_index, outer_position_index, emb_table):
    raise NotImplementedError("write your pallas kernel here")


# speedup vs baseline: 3.2170x; 3.2170x over previous
import functools
import math

import jax
import jax.numpy as jnp
import numpy as np
from jax import lax
from jax.experimental import pallas as pl
from jax.experimental.pallas import tpu as pltpu
from jax.experimental.pallas import tpu_sc as plsc

EMB = 64
HALF = 32
VOCAB = 100000
MAX_LEN = 5000
B, S = 4096, 200
N = B * S
N2 = 2 * N
NC, NS = 2, 16
NW = NC * NS
PER_W = N2 // NW
CHUNK = 1024
ROUNDS = PER_W // CHUNK
SUB = 128
NSUB = CHUNK // SUB


def _build_pe_np():
    position = np.arange(0, MAX_LEN, dtype=np.float32)[:, None]
    div_term = np.exp(
        np.arange(0, HALF, 2, dtype=np.float32) * -(math.log(10000.0) / HALF))
    pe = np.zeros((MAX_LEN, HALF), dtype=np.float32)
    pe[:, 0::2] = np.sin(position * div_term)
    pe[:, 1::2] = np.cos(position * div_term)
    return pe


_PE = _build_pe_np()


def _scale_body(x_ref, o_ref):
    o_ref[...] = x_ref[...] * 8.0


def _scale_table(emb_table):
    flat = emb_table.reshape(VOCAB * EMB // 128, 128)
    rows = flat.shape[0]
    blk = 1000
    out = pl.pallas_call(
        _scale_body,
        out_shape=jax.ShapeDtypeStruct((rows, 128), jnp.float32),
        grid=(rows // blk,),
        in_specs=[pl.BlockSpec((blk, 128), lambda i: (i, 0))],
        out_specs=pl.BlockSpec((blk, 128), lambda i: (i, 0)),
    )(flat)
    return out.reshape(2 * VOCAB, HALF)


_mesh = plsc.VectorSubcoreMesh(core_axis_name="c", subcore_axis_name="s")


@functools.partial(
    pl.kernel,
    mesh=_mesh,
    out_type=jax.ShapeDtypeStruct((N2, HALF), jnp.float32),
    scratch_types=[
        pltpu.VMEM((NSUB, SUB), jnp.int32),
        pltpu.VMEM((NSUB, SUB), jnp.int32),
        pltpu.VMEM((CHUNK, HALF), jnp.float32),
        pltpu.SemaphoreType.DMA,
    ],
    compiler_params=pltpu.CompilerParams(use_tc_tiling_on_sc=False),
)
def _sc_gather(emb32_hbm, pe_hbm, idxe_hbm, idxp_hbm, out_hbm,
               idxe_v, idxp_v, rows_v, sem):
    wid = lax.axis_index("s") * NC + lax.axis_index("c")
    wbase = wid * PER_W

    def round_body(r, carry):
        base = pl.multiple_of(wbase + r * CHUNK, CHUNK)
        irow = pl.multiple_of(base // SUB, 8)
        pltpu.sync_copy(idxe_hbm.at[pl.ds(irow, NSUB)], idxe_v)
        pltpu.sync_copy(idxp_hbm.at[pl.ds(irow, NSUB)], idxp_v)
        handles = []
        for j in range(NSUB):
            handles.append(pltpu.async_copy(
                emb32_hbm.at[idxe_v.at[j]],
                rows_v.at[pl.ds(j * SUB, SUB)],
                sem,
            ))
        for h in handles:
            h.wait()
        handles = []
        for j in range(NSUB):
            handles.append(pltpu.async_copy(
                pe_hbm.at[idxp_v.at[j]],
                rows_v.at[pl.ds(j * SUB, SUB)],
                sem,
                add=True,
            ))
        for h in handles:
            h.wait()
        pltpu.sync_copy(rows_v, out_hbm.at[pl.ds(base, CHUNK)])
        return carry

    lax.fori_loop(0, ROUNDS, round_body, 0)


def kernel(x, inner_position_index, outer_position_index, emb_table):
    x = x.reshape(-1).astype(jnp.int32)
    inner = inner_position_index.reshape(-1).astype(jnp.int32)
    outer = outer_position_index.reshape(-1).astype(jnp.int32)

    emb32 = _scale_table(emb_table)
    pe = jnp.asarray(_PE)

    x2 = 2 * x
    idx_emb = jnp.stack([x2, x2 + 1], axis=-1).reshape(N2 // SUB, SUB)
    idx_pe = jnp.stack([inner, outer], axis=-1).reshape(N2 // SUB, SUB)

    out32 = _sc_gather(emb32, pe, idx_emb, idx_pe)
    return out32.reshape(B, S, EMB)
